# baseline (speedup 1.0000x reference)
import functools

import jax
import jax.numpy as jnp
from jax import lax
from jax.experimental import pallas as pl
from jax.experimental.pallas import tpu as pltpu
from jax.experimental.pallas import tpu_sc as plsc

NUM_EXPERTS = 8
TOP_K = 2
HIDDEN = 1024
FFN = 3584
TOKENS = 2048
ASSIGN = TOKENS * TOP_K
BLK = 256
MAX_BLOCKS = ASSIGN // BLK + NUM_EXPERTS
P_MAX = MAX_BLOCKS * BLK


def _router_kernel(x_ref, gw_ref, logits_ref, w0_ref, w1_ref,
                   pos0_ref, pos1_ref, meta_ref):
    x = x_ref[...]
    logits = lax.dot_general(x, gw_ref[...], (((1,), (1,)), ((), ())),
                             preferred_element_type=jnp.float32)
    logits_ref[...] = logits
    p = jax.nn.softmax(logits, axis=-1)
    cols = lax.broadcasted_iota(jnp.int32, p.shape, 1)
    i1 = jnp.argmax(p, axis=-1)
    sel1 = cols == i1[:, None]
    p1 = jnp.sum(jnp.where(sel1, p, 0.0), axis=-1, keepdims=True)
    pm = jnp.where(sel1, -jnp.inf, p)
    i2 = jnp.argmax(pm, axis=-1)
    sel2 = cols == i2[:, None]
    p2 = jnp.sum(jnp.where(sel2, p, 0.0), axis=-1, keepdims=True)
    denom = p1 + p2
    w0_ref[...] = p1 / denom
    w1_ref[...] = p2 / denom

    n = (jnp.logical_or(sel1, sel2)).astype(jnp.float32)
    r_io = lax.broadcasted_iota(jnp.int32, (TOKENS, TOKENS), 0)
    c_io = lax.broadcasted_iota(jnp.int32, (TOKENS, TOKENS), 1)
    slt = (c_io < r_io).astype(jnp.float32)
    csum = lax.dot_general(slt, n, (((1,), (0,)), ((), ())),
                           preferred_element_type=jnp.float32)

    ones_t = jnp.ones((TOKENS, 1), jnp.float32)
    counts = lax.dot_general(n, ones_t, (((0,), (0,)), ((), ())),
                             preferred_element_type=jnp.float32)
    blocks = jnp.floor((counts + (BLK - 1)) * (1.0 / BLK))
    e_r = lax.broadcasted_iota(jnp.int32, (NUM_EXPERTS, NUM_EXPERTS), 0)
    e_c = lax.broadcasted_iota(jnp.int32, (NUM_EXPERTS, NUM_EXPERTS), 1)
    lt_incl = (e_c <= e_r).astype(jnp.float32)
    cum_incl = lax.dot_general(lt_incl, blocks, (((1,), (0,)), ((), ())),
                               preferred_element_type=jnp.float32)
    off = (cum_incl - blocks) * float(BLK)

    base0 = lax.dot_general(sel1.astype(jnp.float32), off,
                            (((1,), (0,)), ((), ())),
                            preferred_element_type=jnp.float32)
    base1 = lax.dot_general(sel2.astype(jnp.float32), off,
                            (((1,), (0,)), ((), ())),
                            preferred_element_type=jnp.float32)
    c0 = jnp.sum(jnp.where(sel1, csum, 0.0), axis=-1, keepdims=True)
    c1 = jnp.sum(jnp.where(sel2, csum, 0.0), axis=-1, keepdims=True)
    pos0_ref[...] = (base0 + c0).astype(jnp.int32)
    pos1_ref[...] = (base1 + c1).astype(jnp.int32)
    total_blocks = lax.dot_general(
        jnp.ones((1, NUM_EXPERTS), jnp.float32), blocks,
        (((1,), (0,)), ((), ())), preferred_element_type=jnp.float32)
    b_io = lax.broadcasted_iota(jnp.int32, (NUM_EXPERTS, 128), 1
                                ).astype(jnp.float32)
    ge = (b_io >= cum_incl).astype(jnp.float32)
    be_row = jnp.sum(ge, axis=0, keepdims=True)
    act_row = (lax.broadcasted_iota(jnp.int32, (1, 128), 1
                                    ).astype(jnp.float32)
               < total_blocks).astype(jnp.float32)
    meta_ref[0:1, :] = be_row.astype(jnp.int32)
    meta_ref[1:2, :] = act_row.astype(jnp.int32)


def _run_router(x32, gate_w):
    outs = pl.pallas_call(
        _router_kernel,
        grid=(1,),
        in_specs=[
            pl.BlockSpec((TOKENS, HIDDEN), lambda i: (0, 0)),
            pl.BlockSpec((NUM_EXPERTS, HIDDEN), lambda i: (0, 0)),
        ],
        out_specs=[
            pl.BlockSpec((TOKENS, NUM_EXPERTS), lambda i: (0, 0)),
            pl.BlockSpec((TOKENS, 1), lambda i: (0, 0)),
            pl.BlockSpec((TOKENS, 1), lambda i: (0, 0)),
            pl.BlockSpec((TOKENS, 1), lambda i: (0, 0)),
            pl.BlockSpec((TOKENS, 1), lambda i: (0, 0)),
            pl.BlockSpec((8, 128), lambda i: (0, 0)),
        ],
        out_shape=[
            jax.ShapeDtypeStruct((TOKENS, NUM_EXPERTS), jnp.float32),
            jax.ShapeDtypeStruct((TOKENS, 1), jnp.float32),
            jax.ShapeDtypeStruct((TOKENS, 1), jnp.float32),
            jax.ShapeDtypeStruct((TOKENS, 1), jnp.int32),
            jax.ShapeDtypeStruct((TOKENS, 1), jnp.int32),
            jax.ShapeDtypeStruct((8, 128), jnp.int32),
        ],
    )(x32, gate_w)
    return outs


def _make_permute_kernel():
    mesh = plsc.VectorSubcoreMesh(core_axis_name="c", subcore_axis_name="s")

    @functools.partial(
        pl.kernel, mesh=mesh,
        out_type=[
            jax.ShapeDtypeStruct((P_MAX,), jnp.int32),
            jax.ShapeDtypeStruct((P_MAX,), jnp.float32),
        ],
        scratch_types=[
            pltpu.VMEM((ASSIGN,), jnp.int32),
            pltpu.VMEM((ASSIGN,), jnp.float32),
            pltpu.VMEM((P_MAX,), jnp.int32),
            pltpu.VMEM((P_MAX,), jnp.float32),
        ],
        compiler_params=pltpu.CompilerParams(needs_layout_passes=False),
    )
    def permute(pos_hbm, w_hbm, tid_out, ws_out, pos_v, w_v, tid_v, ws_v):
        wid = lax.axis_index("s") * 2 + lax.axis_index("c")

        @pl.when(wid == 0)
        def _():
            pltpu.sync_copy(pos_hbm, pos_v)
            pltpu.sync_copy(w_hbm, w_v)
            zero_i = jnp.zeros((16,), jnp.int32)
            zero_f = jnp.zeros((16,), jnp.float32)

            def init_body(c, _):
                tid_v[pl.ds(c * 16, 16)] = zero_i
                ws_v[pl.ds(c * 16, 16)] = zero_f
                return _
            lax.fori_loop(0, P_MAX // 16, init_body, 0)

            io = lax.iota(jnp.int32, 16)

            def body(c, _):
                base = c * 16
                idx = pos_v[pl.ds(base, 16)]
                wv = w_v[pl.ds(base, 16)]
                tid = lax.shift_right_logical(base + io, 1)
                plsc.store_scatter(tid_v, [idx], tid)
                plsc.store_scatter(ws_v, [idx], wv)
                return _
            lax.fori_loop(0, ASSIGN // 16, body, 0)

            pltpu.sync_copy(tid_v, tid_out)
            pltpu.sync_copy(ws_v, ws_out)

    return permute


def _make_gather_kernel():
    mesh = plsc.VectorSubcoreMesh(core_axis_name="c", subcore_axis_name="s")
    per_w = P_MAX // 32

    @functools.partial(
        pl.kernel, mesh=mesh,
        out_type=jax.ShapeDtypeStruct((P_MAX, HIDDEN // 2), jnp.int32),
        scratch_types=[
            pltpu.VMEM((per_w,), jnp.int32),
            pltpu.VMEM((per_w, HIDDEN // 2), jnp.int32),
            pltpu.SemaphoreType.DMA,
        ],
        compiler_params=pltpu.CompilerParams(needs_layout_passes=False),
    )
    def gather(x_hbm, tid_hbm, xs_out, idx_v, rows_v, sem):
        wid = lax.axis_index("s") * 2 + lax.axis_index("c")
        base = wid * per_w
        pltpu.sync_copy(tid_hbm.at[pl.ds(base, per_w)], idx_v)
        pltpu.async_copy(x_hbm.at[idx_v], rows_v, sem).wait()
        pltpu.sync_copy(rows_v, xs_out.at[pl.ds(base, per_w)])

    return gather


@functools.lru_cache(maxsize=1)
def _sc_kernels():
    return _make_permute_kernel(), _make_gather_kernel()


def _mlp_kernel(be_ref, act_ref, xs_ref, tid_ref, ws_ref,
                w1_ref, w3_ref, w2_ref, final_ref):
    b = pl.program_id(0)

    @pl.when(b == 0)
    def _init():
        final_ref[...] = jnp.zeros_like(final_ref)

    @pl.when(act_ref[b] == 1)
    def _compute():
        xs = xs_ref[...]
        w1 = w1_ref[0]
        w3 = w3_ref[0]
        a = lax.dot_general(xs, w1, (((1,), (1,)), ((), ())),
                            preferred_element_type=jnp.float32)
        g = lax.dot_general(xs, w3, (((1,), (1,)), ((), ())),
                            preferred_element_type=jnp.float32)
        h = jax.nn.silu(a) * g * ws_ref[0]
        hb = h.astype(jnp.bfloat16)
        w2 = w2_ref[0]
        y = lax.dot_general(hb, w2, (((1,), (1,)), ((), ())),
                            preferred_element_type=jnp.float32)
        tid = tid_ref[0]
        cols = lax.broadcasted_iota(jnp.int32, (BLK, TOKENS), 1)
        onehot = (cols == tid).astype(jnp.bfloat16)
        final_ref[...] += lax.dot_general(
            onehot, y.astype(jnp.bfloat16), (((0,), (0,)), ((), ())),
            preferred_element_type=jnp.float32)


def _run_mlp(be, act, xs, tid3, ws3, w1b, w3b, w2b):
    grid_spec = pltpu.PrefetchScalarGridSpec(
        num_scalar_prefetch=2,
        grid=(MAX_BLOCKS,),
        in_specs=[
            pl.BlockSpec((BLK, HIDDEN), lambda b, be_r, act_r: (b, 0)),
            pl.BlockSpec((1, BLK, 1), lambda b, be_r, act_r: (b, 0, 0)),
            pl.BlockSpec((1, BLK, 1), lambda b, be_r, act_r: (b, 0, 0)),
            pl.BlockSpec((1, FFN, HIDDEN),
                         lambda b, be_r, act_r: (be_r[b], 0, 0)),
            pl.BlockSpec((1, FFN, HIDDEN),
                         lambda b, be_r, act_r: (be_r[b], 0, 0)),
            pl.BlockSpec((1, HIDDEN, FFN),
                         lambda b, be_r, act_r: (be_r[b], 0, 0)),
        ],
        out_specs=pl.BlockSpec((TOKENS, HIDDEN),
                               lambda b, be_r, act_r: (0, 0)),
    )
    return pl.pallas_call(
        _mlp_kernel,
        grid_spec=grid_spec,
        out_shape=jax.ShapeDtypeStruct((TOKENS, HIDDEN), jnp.float32),
        compiler_params=pltpu.CompilerParams(
            vmem_limit_bytes=112 * 1024 * 1024),
    )(be, act, xs, tid3, ws3, w1b, w3b, w2b)


@jax.jit
def kernel(hidden_states, gate_w, w1, w2, w3):
    batch, seq, hid = hidden_states.shape
    x32 = hidden_states.reshape(batch * seq, hid)
    w1b = w1.astype(jnp.bfloat16)
    w3b = w3.astype(jnp.bfloat16)
    w2b = w2.astype(jnp.bfloat16)

    logits, wt0, wt1, pos0, pos1, meta = _run_router(x32, gate_w)

    pos_flat = jnp.concatenate([pos0, pos1], axis=1).reshape(ASSIGN)
    w_flat = jnp.concatenate([wt0, wt1], axis=1).reshape(ASSIGN)
    be = meta[0, :MAX_BLOCKS]
    act = meta[1, :MAX_BLOCKS]

    permute_k, gather_k = _sc_kernels()
    tid_sorted, ws_sorted = permute_k(pos_flat, w_flat)
    x_pack = lax.bitcast_convert_type(
        x32.astype(jnp.bfloat16).reshape(TOKENS, HIDDEN // 2, 2), jnp.int32)
    xs_pack = gather_k(x_pack, tid_sorted)
    xs = lax.bitcast_convert_type(xs_pack, jnp.bfloat16).reshape(P_MAX, HIDDEN)

    tid3 = tid_sorted.reshape(MAX_BLOCKS, BLK, 1)
    ws3 = ws_sorted.reshape(MAX_BLOCKS, BLK, 1)
    final = _run_mlp(be, act, xs, tid3, ws3, w1b, w3b, w2b)
    return final.reshape(batch, seq, hid), logits


# speedup vs baseline: 1.0048x; 1.0048x over previous
import functools

import jax
import jax.numpy as jnp
from jax import lax
from jax.experimental import pallas as pl
from jax.experimental.pallas import tpu as pltpu
from jax.experimental.pallas import tpu_sc as plsc

NUM_EXPERTS = 8
TOP_K = 2
HIDDEN = 1024
FFN = 3584
TOKENS = 2048
ASSIGN = TOKENS * TOP_K
BLK = 256
MAX_BLOCKS = ASSIGN // BLK + NUM_EXPERTS
P_MAX = MAX_BLOCKS * BLK


def _router_kernel(x_ref, gw_ref, logits_ref, w0_ref, w1_ref,
                   pos0_ref, pos1_ref, meta_ref, xb_ref):
    x = x_ref[...]
    xb_ref[...] = x.astype(jnp.bfloat16)
    logits = lax.dot_general(x, gw_ref[...], (((1,), (1,)), ((), ())),
                             preferred_element_type=jnp.float32)
    logits_ref[...] = logits
    p = jax.nn.softmax(logits, axis=-1)
    cols = lax.broadcasted_iota(jnp.int32, p.shape, 1)
    i1 = jnp.argmax(p, axis=-1)
    sel1 = cols == i1[:, None]
    p1 = jnp.sum(jnp.where(sel1, p, 0.0), axis=-1, keepdims=True)
    pm = jnp.where(sel1, -jnp.inf, p)
    i2 = jnp.argmax(pm, axis=-1)
    sel2 = cols == i2[:, None]
    p2 = jnp.sum(jnp.where(sel2, p, 0.0), axis=-1, keepdims=True)
    denom = p1 + p2
    w0_ref[...] = p1 / denom
    w1_ref[...] = p2 / denom

    n = (jnp.logical_or(sel1, sel2)).astype(jnp.float32)
    r_io = lax.broadcasted_iota(jnp.int32, (TOKENS, TOKENS), 0)
    c_io = lax.broadcasted_iota(jnp.int32, (TOKENS, TOKENS), 1)
    slt = (c_io < r_io).astype(jnp.float32)
    csum = lax.dot_general(slt, n, (((1,), (0,)), ((), ())),
                           preferred_element_type=jnp.float32)

    ones_t = jnp.ones((TOKENS, 1), jnp.float32)
    counts = lax.dot_general(n, ones_t, (((0,), (0,)), ((), ())),
                             preferred_element_type=jnp.float32)
    blocks = jnp.floor((counts + (BLK - 1)) * (1.0 / BLK))
    e_r = lax.broadcasted_iota(jnp.int32, (NUM_EXPERTS, NUM_EXPERTS), 0)
    e_c = lax.broadcasted_iota(jnp.int32, (NUM_EXPERTS, NUM_EXPERTS), 1)
    lt_incl = (e_c <= e_r).astype(jnp.float32)
    cum_incl = lax.dot_general(lt_incl, blocks, (((1,), (0,)), ((), ())),
                               preferred_element_type=jnp.float32)
    off = (cum_incl - blocks) * float(BLK)

    base0 = lax.dot_general(sel1.astype(jnp.float32), off,
                            (((1,), (0,)), ((), ())),
                            preferred_element_type=jnp.float32)
    base1 = lax.dot_general(sel2.astype(jnp.float32), off,
                            (((1,), (0,)), ((), ())),
                            preferred_element_type=jnp.float32)
    c0 = jnp.sum(jnp.where(sel1, csum, 0.0), axis=-1, keepdims=True)
    c1 = jnp.sum(jnp.where(sel2, csum, 0.0), axis=-1, keepdims=True)
    pos0_ref[...] = (base0 + c0).astype(jnp.int32)
    pos1_ref[...] = (base1 + c1).astype(jnp.int32)
    total_blocks = lax.dot_general(
        jnp.ones((1, NUM_EXPERTS), jnp.float32), blocks,
        (((1,), (0,)), ((), ())), preferred_element_type=jnp.float32)
    b_io = lax.broadcasted_iota(jnp.int32, (NUM_EXPERTS, 128), 1
                                ).astype(jnp.float32)
    ge = (b_io >= cum_incl).astype(jnp.float32)
    be_row = jnp.sum(ge, axis=0, keepdims=True)
    act_row = (lax.broadcasted_iota(jnp.int32, (1, 128), 1
                                    ).astype(jnp.float32)
               < total_blocks).astype(jnp.float32)
    meta_ref[0:1, :] = be_row.astype(jnp.int32)
    meta_ref[1:2, :] = act_row.astype(jnp.int32)


def _run_router(x32, gate_w):
    outs = pl.pallas_call(
        _router_kernel,
        grid=(1,),
        in_specs=[
            pl.BlockSpec((TOKENS, HIDDEN), lambda i: (0, 0)),
            pl.BlockSpec((NUM_EXPERTS, HIDDEN), lambda i: (0, 0)),
        ],
        out_specs=[
            pl.BlockSpec((TOKENS, NUM_EXPERTS), lambda i: (0, 0)),
            pl.BlockSpec((TOKENS, 1), lambda i: (0, 0)),
            pl.BlockSpec((TOKENS, 1), lambda i: (0, 0)),
            pl.BlockSpec((TOKENS, 1), lambda i: (0, 0)),
            pl.BlockSpec((TOKENS, 1), lambda i: (0, 0)),
            pl.BlockSpec((8, 128), lambda i: (0, 0)),
            pl.BlockSpec((TOKENS, HIDDEN), lambda i: (0, 0)),
        ],
        out_shape=[
            jax.ShapeDtypeStruct((TOKENS, NUM_EXPERTS), jnp.float32),
            jax.ShapeDtypeStruct((TOKENS, 1), jnp.float32),
            jax.ShapeDtypeStruct((TOKENS, 1), jnp.float32),
            jax.ShapeDtypeStruct((TOKENS, 1), jnp.int32),
            jax.ShapeDtypeStruct((TOKENS, 1), jnp.int32),
            jax.ShapeDtypeStruct((8, 128), jnp.int32),
            jax.ShapeDtypeStruct((TOKENS, HIDDEN), jnp.bfloat16),
        ],
    )(x32, gate_w)
    return outs


def _make_permute_kernel():
    mesh = plsc.VectorSubcoreMesh(core_axis_name="c", subcore_axis_name="s")

    @functools.partial(
        pl.kernel, mesh=mesh,
        out_type=[
            jax.ShapeDtypeStruct((P_MAX,), jnp.int32),
            jax.ShapeDtypeStruct((P_MAX,), jnp.float32),
        ],
        scratch_types=[
            pltpu.VMEM((ASSIGN,), jnp.int32),
            pltpu.VMEM((ASSIGN,), jnp.float32),
            pltpu.VMEM((P_MAX,), jnp.int32),
            pltpu.VMEM((P_MAX,), jnp.float32),
        ],
        compiler_params=pltpu.CompilerParams(needs_layout_passes=False),
    )
    def permute(pos_hbm, w_hbm, tid_out, ws_out, pos_v, w_v, tid_v, ws_v):
        wid = lax.axis_index("s") * 2 + lax.axis_index("c")

        @pl.when(wid == 0)
        def _():
            pltpu.sync_copy(pos_hbm, pos_v)
            pltpu.sync_copy(w_hbm, w_v)
            zero_i = jnp.zeros((16,), jnp.int32)
            zero_f = jnp.zeros((16,), jnp.float32)

            def init_body(c, _):
                tid_v[pl.ds(c * 16, 16)] = zero_i
                ws_v[pl.ds(c * 16, 16)] = zero_f
                return _
            lax.fori_loop(0, P_MAX // 16, init_body, 0)

            io = lax.iota(jnp.int32, 16)

            def body(c, _):
                base = c * 16
                idx = pos_v[pl.ds(base, 16)]
                wv = w_v[pl.ds(base, 16)]
                tid = lax.shift_right_logical(base + io, 1)
                plsc.store_scatter(tid_v, [idx], tid)
                plsc.store_scatter(ws_v, [idx], wv)
                return _
            lax.fori_loop(0, ASSIGN // 16, body, 0)

            pltpu.sync_copy(tid_v, tid_out)
            pltpu.sync_copy(ws_v, ws_out)

    return permute


def _make_gather_kernel():
    mesh = plsc.VectorSubcoreMesh(core_axis_name="c", subcore_axis_name="s")
    per_w = P_MAX // 32

    @functools.partial(
        pl.kernel, mesh=mesh,
        out_type=jax.ShapeDtypeStruct((P_MAX, HIDDEN // 2), jnp.int32),
        scratch_types=[
            pltpu.VMEM((per_w,), jnp.int32),
            pltpu.VMEM((per_w, HIDDEN // 2), jnp.int32),
            pltpu.SemaphoreType.DMA,
        ],
        compiler_params=pltpu.CompilerParams(needs_layout_passes=False),
    )
    def gather(x_hbm, tid_hbm, xs_out, idx_v, rows_v, sem):
        wid = lax.axis_index("s") * 2 + lax.axis_index("c")
        base = wid * per_w
        pltpu.sync_copy(tid_hbm.at[pl.ds(base, per_w)], idx_v)
        pltpu.async_copy(x_hbm.at[idx_v], rows_v, sem).wait()
        pltpu.sync_copy(rows_v, xs_out.at[pl.ds(base, per_w)])

    return gather


@functools.lru_cache(maxsize=1)
def _sc_kernels():
    return _make_permute_kernel(), _make_gather_kernel()


def _mlp_kernel(be_ref, act_ref, xs_ref, tid_ref, ws_ref,
                w1_ref, w3_ref, w2_ref, final_ref):
    b = pl.program_id(0)

    @pl.when(b == 0)
    def _init():
        final_ref[...] = jnp.zeros_like(final_ref)

    @pl.when(act_ref[b] == 1)
    def _compute():
        xs = xs_ref[...]
        w1 = w1_ref[0]
        w3 = w3_ref[0]
        a = lax.dot_general(xs, w1, (((1,), (1,)), ((), ())),
                            preferred_element_type=jnp.float32)
        g = lax.dot_general(xs, w3, (((1,), (1,)), ((), ())),
                            preferred_element_type=jnp.float32)
        h = jax.nn.silu(a) * g * ws_ref[0]
        hb = h.astype(jnp.bfloat16)
        w2 = w2_ref[0]
        y = lax.dot_general(hb, w2, (((1,), (1,)), ((), ())),
                            preferred_element_type=jnp.float32)
        tid = tid_ref[0]
        cols = lax.broadcasted_iota(jnp.int32, (BLK, TOKENS), 1)
        onehot = (cols == tid).astype(jnp.bfloat16)
        final_ref[...] += lax.dot_general(
            onehot, y.astype(jnp.bfloat16), (((0,), (0,)), ((), ())),
            preferred_element_type=jnp.float32)


def _run_mlp(be, act, xs, tid3, ws3, w1b, w3b, w2b):
    grid_spec = pltpu.PrefetchScalarGridSpec(
        num_scalar_prefetch=2,
        grid=(MAX_BLOCKS,),
        in_specs=[
            pl.BlockSpec((BLK, HIDDEN), lambda b, be_r, act_r: (b, 0)),
            pl.BlockSpec((1, BLK, 1), lambda b, be_r, act_r: (b, 0, 0)),
            pl.BlockSpec((1, BLK, 1), lambda b, be_r, act_r: (b, 0, 0)),
            pl.BlockSpec((1, FFN, HIDDEN),
                         lambda b, be_r, act_r: (be_r[b], 0, 0)),
            pl.BlockSpec((1, FFN, HIDDEN),
                         lambda b, be_r, act_r: (be_r[b], 0, 0)),
            pl.BlockSpec((1, HIDDEN, FFN),
                         lambda b, be_r, act_r: (be_r[b], 0, 0)),
        ],
        out_specs=pl.BlockSpec((TOKENS, HIDDEN),
                               lambda b, be_r, act_r: (0, 0)),
    )
    return pl.pallas_call(
        _mlp_kernel,
        grid_spec=grid_spec,
        out_shape=jax.ShapeDtypeStruct((TOKENS, HIDDEN), jnp.float32),
        compiler_params=pltpu.CompilerParams(
            vmem_limit_bytes=112 * 1024 * 1024),
    )(be, act, xs, tid3, ws3, w1b, w3b, w2b)


@jax.jit
def kernel(hidden_states, gate_w, w1, w2, w3):
    batch, seq, hid = hidden_states.shape
    x32 = hidden_states.reshape(batch * seq, hid)
    w1b = w1.astype(jnp.bfloat16)
    w3b = w3.astype(jnp.bfloat16)
    w2b = w2.astype(jnp.bfloat16)

    logits, wt0, wt1, pos0, pos1, meta, xb = _run_router(x32, gate_w)

    pos_flat = jnp.concatenate([pos0, pos1], axis=1).reshape(ASSIGN)
    w_flat = jnp.concatenate([wt0, wt1], axis=1).reshape(ASSIGN)
    be = meta[0, :MAX_BLOCKS]
    act = meta[1, :MAX_BLOCKS]

    permute_k, gather_k = _sc_kernels()
    tid_sorted, ws_sorted = permute_k(pos_flat, w_flat)
    x_pack = lax.bitcast_convert_type(
        xb.reshape(TOKENS, HIDDEN // 2, 2), jnp.int32)
    xs_pack = gather_k(x_pack, tid_sorted)
    xs = lax.bitcast_convert_type(xs_pack, jnp.bfloat16).reshape(P_MAX, HIDDEN)

    tid3 = tid_sorted.reshape(MAX_BLOCKS, BLK, 1)
    ws3 = ws_sorted.reshape(MAX_BLOCKS, BLK, 1)
    final = _run_mlp(be, act, xs, tid3, ws3, w1b, w3b, w2b)
    return final.reshape(batch, seq, hid), logits


# speedup vs baseline: 1.6801x; 1.6721x over previous
import functools

import jax
import jax.numpy as jnp
from jax import lax
from jax.experimental import pallas as pl
from jax.experimental.pallas import tpu as pltpu
from jax.experimental.pallas import tpu_sc as plsc

NUM_EXPERTS = 8
TOP_K = 2
HIDDEN = 1024
FFN = 3584
TOKENS = 2048
ASSIGN = TOKENS * TOP_K
BLK = 256
MAX_BLOCKS = ASSIGN // BLK + NUM_EXPERTS
P_MAX = MAX_BLOCKS * BLK


def _router_kernel(x_ref, gw_ref, logits_ref, w0_ref, w1_ref,
                   pos0_ref, pos1_ref, meta_ref, xb_ref):
    x = x_ref[...]
    xb_ref[...] = x.astype(jnp.bfloat16)
    logits = lax.dot_general(x, gw_ref[...], (((1,), (1,)), ((), ())),
                             preferred_element_type=jnp.float32)
    logits_ref[...] = logits
    p = jax.nn.softmax(logits, axis=-1)
    cols = lax.broadcasted_iota(jnp.int32, p.shape, 1)
    i1 = jnp.argmax(p, axis=-1)
    sel1 = cols == i1[:, None]
    p1 = jnp.sum(jnp.where(sel1, p, 0.0), axis=-1, keepdims=True)
    pm = jnp.where(sel1, -jnp.inf, p)
    i2 = jnp.argmax(pm, axis=-1)
    sel2 = cols == i2[:, None]
    p2 = jnp.sum(jnp.where(sel2, p, 0.0), axis=-1, keepdims=True)
    denom = p1 + p2
    w0_ref[...] = p1 / denom
    w1_ref[...] = p2 / denom

    n = (jnp.logical_or(sel1, sel2)).astype(jnp.float32)
    r_io = lax.broadcasted_iota(jnp.int32, (TOKENS, TOKENS), 0)
    c_io = lax.broadcasted_iota(jnp.int32, (TOKENS, TOKENS), 1)
    slt = (c_io < r_io).astype(jnp.float32)
    csum = lax.dot_general(slt, n, (((1,), (0,)), ((), ())),
                           preferred_element_type=jnp.float32)

    ones_t = jnp.ones((TOKENS, 1), jnp.float32)
    counts = lax.dot_general(n, ones_t, (((0,), (0,)), ((), ())),
                             preferred_element_type=jnp.float32)
    blocks = jnp.floor((counts + (BLK - 1)) * (1.0 / BLK))
    e_r = lax.broadcasted_iota(jnp.int32, (NUM_EXPERTS, NUM_EXPERTS), 0)
    e_c = lax.broadcasted_iota(jnp.int32, (NUM_EXPERTS, NUM_EXPERTS), 1)
    lt_incl = (e_c <= e_r).astype(jnp.float32)
    cum_incl = lax.dot_general(lt_incl, blocks, (((1,), (0,)), ((), ())),
                               preferred_element_type=jnp.float32)
    off = (cum_incl - blocks) * float(BLK)

    base0 = lax.dot_general(sel1.astype(jnp.float32), off,
                            (((1,), (0,)), ((), ())),
                            preferred_element_type=jnp.float32)
    base1 = lax.dot_general(sel2.astype(jnp.float32), off,
                            (((1,), (0,)), ((), ())),
                            preferred_element_type=jnp.float32)
    c0 = jnp.sum(jnp.where(sel1, csum, 0.0), axis=-1, keepdims=True)
    c1 = jnp.sum(jnp.where(sel2, csum, 0.0), axis=-1, keepdims=True)
    pos0_ref[...] = (base0 + c0).astype(jnp.int32)
    pos1_ref[...] = (base1 + c1).astype(jnp.int32)
    total_blocks = lax.dot_general(
        jnp.ones((1, NUM_EXPERTS), jnp.float32), blocks,
        (((1,), (0,)), ((), ())), preferred_element_type=jnp.float32)
    b_io = lax.broadcasted_iota(jnp.int32, (NUM_EXPERTS, 128), 1
                                ).astype(jnp.float32)
    ge = (b_io >= cum_incl).astype(jnp.float32)
    be_row = jnp.sum(ge, axis=0, keepdims=True)
    act_row = (lax.broadcasted_iota(jnp.int32, (1, 128), 1
                                    ).astype(jnp.float32)
               < total_blocks).astype(jnp.float32)
    meta_ref[0:1, :] = be_row.astype(jnp.int32)
    meta_ref[1:2, :] = act_row.astype(jnp.int32)


def _run_router(x32, gate_w):
    outs = pl.pallas_call(
        _router_kernel,
        grid=(1,),
        in_specs=[
            pl.BlockSpec((TOKENS, HIDDEN), lambda i: (0, 0)),
            pl.BlockSpec((NUM_EXPERTS, HIDDEN), lambda i: (0, 0)),
        ],
        out_specs=[
            pl.BlockSpec((TOKENS, NUM_EXPERTS), lambda i: (0, 0)),
            pl.BlockSpec((TOKENS, 1), lambda i: (0, 0)),
            pl.BlockSpec((TOKENS, 1), lambda i: (0, 0)),
            pl.BlockSpec((TOKENS, 1), lambda i: (0, 0)),
            pl.BlockSpec((TOKENS, 1), lambda i: (0, 0)),
            pl.BlockSpec((8, 128), lambda i: (0, 0)),
            pl.BlockSpec((TOKENS, HIDDEN), lambda i: (0, 0)),
        ],
        out_shape=[
            jax.ShapeDtypeStruct((TOKENS, NUM_EXPERTS), jnp.float32),
            jax.ShapeDtypeStruct((TOKENS, 1), jnp.float32),
            jax.ShapeDtypeStruct((TOKENS, 1), jnp.float32),
            jax.ShapeDtypeStruct((TOKENS, 1), jnp.int32),
            jax.ShapeDtypeStruct((TOKENS, 1), jnp.int32),
            jax.ShapeDtypeStruct((8, 128), jnp.int32),
            jax.ShapeDtypeStruct((TOKENS, HIDDEN), jnp.bfloat16),
        ],
    )(x32, gate_w)
    return outs


def _make_permute_kernel():
    mesh = plsc.VectorSubcoreMesh(core_axis_name="c", subcore_axis_name="s")

    @functools.partial(
        pl.kernel, mesh=mesh,
        out_type=[
            jax.ShapeDtypeStruct((P_MAX,), jnp.int32),
            jax.ShapeDtypeStruct((P_MAX,), jnp.float32),
        ],
        scratch_types=[
            pltpu.VMEM((ASSIGN,), jnp.int32),
            pltpu.VMEM((ASSIGN,), jnp.float32),
            pltpu.VMEM((P_MAX,), jnp.int32),
            pltpu.VMEM((P_MAX,), jnp.float32),
        ],
        compiler_params=pltpu.CompilerParams(needs_layout_passes=False),
    )
    def permute(pos_hbm, w_hbm, tid_out, ws_out, pos_v, w_v, tid_v, ws_v):
        wid = lax.axis_index("s") * 2 + lax.axis_index("c")

        @pl.when(wid == 0)
        def _():
            pltpu.sync_copy(pos_hbm, pos_v)
            pltpu.sync_copy(w_hbm, w_v)
            zero_i = jnp.zeros((16,), jnp.int32)
            zero_f = jnp.zeros((16,), jnp.float32)

            def init_body(c, _):
                tid_v[pl.ds(c * 16, 16)] = zero_i
                ws_v[pl.ds(c * 16, 16)] = zero_f
                return _
            lax.fori_loop(0, P_MAX // 16, init_body, 0)

            io = lax.iota(jnp.int32, 16)

            def body(c, _):
                base = c * 16
                idx = pos_v[pl.ds(base, 16)]
                wv = w_v[pl.ds(base, 16)]
                tid = lax.shift_right_logical(base + io, 1)
                plsc.store_scatter(tid_v, [idx], tid)
                plsc.store_scatter(ws_v, [idx], wv)
                return _
            lax.fori_loop(0, ASSIGN // 16, body, 0)

            pltpu.sync_copy(tid_v, tid_out)
            pltpu.sync_copy(ws_v, ws_out)

    return permute


_GCH = 96


def _make_gather_kernel():
    mesh = plsc.VectorSubcoreMesh(core_axis_name="c", subcore_axis_name="s")
    per_w = P_MAX // 32
    SL = HIDDEN // 128

    @functools.partial(
        pl.kernel, mesh=mesh,
        out_type=jax.ShapeDtypeStruct((P_MAX, SL, 128), jnp.bfloat16),
        scratch_types=[
            pltpu.VMEM((_GCH,), jnp.int32),
            pltpu.VMEM((_GCH,), jnp.int32),
            pltpu.VMEM((_GCH, SL, 128), jnp.bfloat16),
            pltpu.VMEM((_GCH, SL, 128), jnp.bfloat16),
            pltpu.SemaphoreType.DMA,
            pltpu.SemaphoreType.DMA,
        ],
        compiler_params=pltpu.CompilerParams(
            needs_layout_passes=False, use_tc_tiling_on_sc=True),
    )
    def gather(x_hbm, tid_hbm, xs_out, idx0, idx1, rows0, rows1, sem0, sem1):
        wid = lax.axis_index("s") * 2 + lax.axis_index("c")
        base = wid * per_w
        pltpu.sync_copy(tid_hbm.at[pl.ds(base, _GCH)], idx0)
        pltpu.sync_copy(tid_hbm.at[pl.ds(base + _GCH, _GCH)], idx1)
        c0 = pltpu.async_copy(x_hbm.at[idx0], rows0, sem0)
        c1 = pltpu.async_copy(x_hbm.at[idx1], rows1, sem1)
        c0.wait()
        pltpu.sync_copy(rows0, xs_out.at[pl.ds(base, _GCH)])
        c1.wait()
        pltpu.sync_copy(rows1, xs_out.at[pl.ds(base + _GCH, _GCH)])

    return gather


@functools.lru_cache(maxsize=1)
def _sc_kernels():
    return _make_permute_kernel(), _make_gather_kernel()


def _mlp_kernel(be_ref, act_ref, xb_ref, tid_ref, ws_ref,
                w1_ref, w3_ref, w2_ref, final_ref):
    b = pl.program_id(0)

    @pl.when(b == 0)
    def _init():
        final_ref[...] = jnp.zeros_like(final_ref)

    @pl.when(act_ref[b] == 1)
    def _compute():
        tid = tid_ref[0]
        cols = lax.broadcasted_iota(jnp.int32, (BLK, TOKENS), 1)
        onehot = (cols == tid).astype(jnp.bfloat16)
        xs = lax.dot_general(onehot, xb_ref[...], (((1,), (0,)), ((), ())),
                             preferred_element_type=jnp.float32
                             ).astype(jnp.bfloat16)
        w1 = w1_ref[0]
        w3 = w3_ref[0]
        a = lax.dot_general(xs, w1, (((1,), (1,)), ((), ())),
                            preferred_element_type=jnp.float32)
        g = lax.dot_general(xs, w3, (((1,), (1,)), ((), ())),
                            preferred_element_type=jnp.float32)
        h = jax.nn.silu(a) * g * ws_ref[0]
        hb = h.astype(jnp.bfloat16)
        w2 = w2_ref[0]
        y = lax.dot_general(hb, w2, (((1,), (1,)), ((), ())),
                            preferred_element_type=jnp.float32)
        final_ref[...] += lax.dot_general(
            onehot, y.astype(jnp.bfloat16), (((0,), (0,)), ((), ())),
            preferred_element_type=jnp.float32)


def _run_mlp(be, act, xb, tid3, ws3, w1b, w3b, w2b):
    grid_spec = pltpu.PrefetchScalarGridSpec(
        num_scalar_prefetch=2,
        grid=(MAX_BLOCKS,),
        in_specs=[
            pl.BlockSpec((TOKENS, HIDDEN), lambda b, be_r, act_r: (0, 0)),
            pl.BlockSpec((1, BLK, 1), lambda b, be_r, act_r: (b, 0, 0)),
            pl.BlockSpec((1, BLK, 1), lambda b, be_r, act_r: (b, 0, 0)),
            pl.BlockSpec((1, FFN, HIDDEN),
                         lambda b, be_r, act_r: (be_r[b], 0, 0)),
            pl.BlockSpec((1, FFN, HIDDEN),
                         lambda b, be_r, act_r: (be_r[b], 0, 0)),
            pl.BlockSpec((1, HIDDEN, FFN),
                         lambda b, be_r, act_r: (be_r[b], 0, 0)),
        ],
        out_specs=pl.BlockSpec((TOKENS, HIDDEN),
                               lambda b, be_r, act_r: (0, 0)),
    )
    return pl.pallas_call(
        _mlp_kernel,
        grid_spec=grid_spec,
        out_shape=jax.ShapeDtypeStruct((TOKENS, HIDDEN), jnp.float32),
        compiler_params=pltpu.CompilerParams(
            vmem_limit_bytes=112 * 1024 * 1024),
    )(be, act, xb, tid3, ws3, w1b, w3b, w2b)


@jax.jit
def kernel(hidden_states, gate_w, w1, w2, w3):
    batch, seq, hid = hidden_states.shape
    x32 = hidden_states.reshape(batch * seq, hid)
    w1b = w1.astype(jnp.bfloat16)
    w3b = w3.astype(jnp.bfloat16)
    w2b = w2.astype(jnp.bfloat16)

    logits, wt0, wt1, pos0, pos1, meta, xb = _run_router(x32, gate_w)

    pos_flat = jnp.concatenate([pos0, pos1], axis=1).reshape(ASSIGN)
    w_flat = jnp.concatenate([wt0, wt1], axis=1).reshape(ASSIGN)
    be = meta[0, :MAX_BLOCKS]
    act = meta[1, :MAX_BLOCKS]

    permute_k = _sc_kernels()[0]
    tid_sorted, ws_sorted = permute_k(pos_flat, w_flat)

    tid3 = tid_sorted.reshape(MAX_BLOCKS, BLK, 1)
    ws3 = ws_sorted.reshape(MAX_BLOCKS, BLK, 1)
    final = _run_mlp(be, act, xb, tid3, ws3, w1b, w3b, w2b)
    return final.reshape(batch, seq, hid), logits
